# baseline (device time: 21715 ns/iter reference)
import jax
import jax.numpy as jnp
from jax import lax
from jax.experimental import pallas as pl
from jax.experimental.pallas import tpu as pltpu

M = 1024
NCOL = 512
K = 32
RPC = M // K


def kernel(x):
    def body(x_ref, out_ref, xbuf, rbuf_x, send1, recv1, send2, recv2):
        my_x = lax.axis_index("x")
        my_y = lax.axis_index("y")
        x_nbr = (1 - my_x, my_y)
        y_nbr = (my_x, 1 - my_y)
        my_col = pl.ds(my_y * NCOL, NCOL)

        barrier = pltpu.get_barrier_semaphore()
        for nbr in (x_nbr, y_nbr):
            pl.semaphore_signal(
                barrier, inc=1, device_id=nbr,
                device_id_type=pl.DeviceIdType.MESH,
            )
        pl.semaphore_wait(barrier, 2)

        rdma1 = []
        for c in range(K):
            rows = pl.ds(c * RPC, RPC)
            xbuf[rows, :] = x_ref[0, rows, :].astype(jnp.bfloat16)
            r = pltpu.make_async_remote_copy(
                src_ref=xbuf.at[rows],
                dst_ref=rbuf_x.at[rows],
                send_sem=send1.at[c], recv_sem=recv1.at[c],
                device_id=x_nbr, device_id_type=pl.DeviceIdType.MESH,
            )
            r.start()
            rdma1.append(r)

        rdma2 = []
        for c in range(K):
            rows = pl.ds(c * RPC, RPC)
            rdma1[c].wait_recv()
            s = xbuf[rows, :] + rbuf_x[rows, :]

            @pl.when(my_y == 0)
            def _():
                out_ref[rows, :NCOL] = s

            @pl.when(my_y == 1)
            def _():
                out_ref[rows, NCOL:] = s

            r = pltpu.make_async_remote_copy(
                src_ref=out_ref.at[rows, my_col],
                dst_ref=out_ref.at[rows, my_col],
                send_sem=send2.at[c], recv_sem=recv2.at[c],
                device_id=y_nbr, device_id_type=pl.DeviceIdType.MESH,
            )
            r.start()
            rdma2.append(r)

        for c in range(K):
            rdma2[c].wait_recv()

        for c in range(K):
            rdma1[c].wait_send()
            rdma2[c].wait_send()

    return pl.pallas_call(
        body,
        out_shape=jax.ShapeDtypeStruct((M, 2 * NCOL), jnp.bfloat16),
        in_specs=[pl.BlockSpec(memory_space=pltpu.VMEM)],
        out_specs=pl.BlockSpec(memory_space=pltpu.VMEM),
        scratch_shapes=[
            pltpu.VMEM((M, NCOL), jnp.bfloat16),
            pltpu.VMEM((M, NCOL), jnp.bfloat16),
            pltpu.SemaphoreType.DMA((K,)),
            pltpu.SemaphoreType.DMA((K,)),
            pltpu.SemaphoreType.DMA((K,)),
            pltpu.SemaphoreType.DMA((K,)),
        ],
        compiler_params=pltpu.CompilerParams(collective_id=0),
    )(x)


# device time: 21467 ns/iter; 1.0116x vs baseline; 1.0116x over previous
import jax
import jax.numpy as jnp
from jax import lax
from jax.experimental import pallas as pl
from jax.experimental.pallas import tpu as pltpu

M = 1024
NCOL = 512
K = 16
RPC = M // K


def kernel(x):
    def body(x_ref, out_ref, xbuf, rbuf_x, send1, recv1, send2, recv2):
        my_x = lax.axis_index("x")
        my_y = lax.axis_index("y")
        x_nbr = (1 - my_x, my_y)
        y_nbr = (my_x, 1 - my_y)
        my_col = pl.ds(my_y * NCOL, NCOL)

        barrier = pltpu.get_barrier_semaphore()
        for nbr in (x_nbr, y_nbr):
            pl.semaphore_signal(
                barrier, inc=1, device_id=nbr,
                device_id_type=pl.DeviceIdType.MESH,
            )
        pl.semaphore_wait(barrier, 2)

        rdma1 = []
        for c in range(K):
            rows = pl.ds(c * RPC, RPC)
            xbuf[rows, :] = x_ref[0, rows, :].astype(jnp.bfloat16)
            r = pltpu.make_async_remote_copy(
                src_ref=xbuf.at[rows],
                dst_ref=rbuf_x.at[rows],
                send_sem=send1.at[c], recv_sem=recv1.at[c],
                device_id=x_nbr, device_id_type=pl.DeviceIdType.MESH,
            )
            r.start()
            rdma1.append(r)

        rdma2 = []
        for c in range(K):
            rows = pl.ds(c * RPC, RPC)
            rdma1[c].wait_recv()
            s = xbuf[rows, :] + rbuf_x[rows, :]

            @pl.when(my_y == 0)
            def _():
                out_ref[rows, :NCOL] = s

            @pl.when(my_y == 1)
            def _():
                out_ref[rows, NCOL:] = s

            r = pltpu.make_async_remote_copy(
                src_ref=out_ref.at[rows, my_col],
                dst_ref=out_ref.at[rows, my_col],
                send_sem=send2.at[c], recv_sem=recv2.at[c],
                device_id=y_nbr, device_id_type=pl.DeviceIdType.MESH,
            )
            r.start()
            rdma2.append(r)

        for c in range(K):
            rdma2[c].wait_recv()

        for c in range(K):
            rdma1[c].wait_send()
            rdma2[c].wait_send()

    return pl.pallas_call(
        body,
        out_shape=jax.ShapeDtypeStruct((M, 2 * NCOL), jnp.bfloat16),
        in_specs=[pl.BlockSpec(memory_space=pltpu.VMEM)],
        out_specs=pl.BlockSpec(memory_space=pltpu.VMEM),
        scratch_shapes=[
            pltpu.VMEM((M, NCOL), jnp.bfloat16),
            pltpu.VMEM((M, NCOL), jnp.bfloat16),
            pltpu.SemaphoreType.DMA((K,)),
            pltpu.SemaphoreType.DMA((K,)),
            pltpu.SemaphoreType.DMA((K,)),
            pltpu.SemaphoreType.DMA((K,)),
        ],
        compiler_params=pltpu.CompilerParams(collective_id=0),
    )(x)
